# baseline (speedup 1.0000x reference)
import functools

import jax
import jax.numpy as jnp
from jax import lax
from jax.experimental import pallas as pl
from jax.experimental.pallas import tpu as pltpu
from jax.experimental.pallas import tpu_sc as plsc

_D = 32
_NW = 32
_NB = 256
_NGI = _NB // 128
_TP = _NB + 2


@functools.cache
def _make_kernel(BATCH: int, HIST: int):
    nb_strip = BATCH // _NW
    n_half = nb_strip // _NB
    n_chunk = HIST * n_half
    mesh = plsc.VectorSubcoreMesh(core_axis_name="c", subcore_axis_name="s")

    @functools.partial(
        pl.kernel,
        out_type=jax.ShapeDtypeStruct((HIST, _D, BATCH), jnp.float32),
        mesh=mesh,
        scratch_types=[
            pltpu.VMEM((HIST, nb_strip), jnp.int32),
            pltpu.VMEM((2 * _NGI, 128), jnp.int32),
            pltpu.VMEM((2, _NB), jnp.int32),
            pltpu.VMEM((2, _NB, 128), jnp.float32),
            pltpu.VMEM((2, _D, _TP), jnp.float32),
            pltpu.SemaphoreType.DMA,
            pltpu.SemaphoreType.DMA,
            pltpu.SemaphoreType.DMA,
            pltpu.SemaphoreType.DMA,
        ],
        compiler_params=pltpu.CompilerParams(needs_layout_passes=False),
    )
    def gather_kernel(idx_hbm, table2_hbm, out_hbm,
                      idx_local, gi, qoff, rows_v, trans_v,
                      sem0, sem1, osem0, osem1):
        wid = lax.axis_index("s") * 2 + lax.axis_index("c")
        b0 = wid * nb_strip
        iota16 = lax.iota(jnp.int32, 16)
        sems = (sem0, sem1)
        osems = (osem0, osem1)

        pltpu.sync_copy(idx_hbm.at[:, pl.ds(b0, nb_strip)], idx_local)

        def fetch(c, buf):
            h = c // n_half
            off = (c - h * n_half) * _NB
            for t in range(_NB // 16):
                v = idx_local[h, pl.ds(off + t * 16, 16)]
                gi[buf * _NGI + t // 8, pl.ds((t % 8) * 16, 16)] = (
                    lax.shift_left(lax.shift_right_logical(v, 9), 7)
                    + jnp.bitwise_and(v, 127))
                qoff[buf, pl.ds(t * 16, 16)] = lax.shift_left(
                    jnp.bitwise_and(lax.shift_right_logical(v, 7), 3), 5)
            for j in range(_NGI):
                pltpu.async_copy(
                    table2_hbm.at[gi.at[buf * _NGI + j]],
                    rows_v.at[buf, pl.ds(j * 128, 128)],
                    sems[buf],
                )

        def drain(buf):
            for j in range(_NGI):
                pltpu.make_async_copy(
                    table2_hbm.at[gi.at[buf * _NGI + j]],
                    rows_v.at[buf, pl.ds(j * 128, 128)],
                    sems[buf],
                ).wait()

        skews = [jnp.bitwise_and(dd + iota16, 15) for dd in range(16)]

        def emit(c, buf):
            def block(bg, carry):
                base = bg * 16
                row_ids = base + iota16
                qv = qoff[buf, pl.ds(base, 16)]
                for d0 in range(0, _D, 16):
                    for dd in range(16):
                        vals = plsc.load_gather(
                            rows_v.at[buf], [row_ids, qv + (d0 + skews[dd])])
                        plsc.store_scatter(
                            trans_v.at[buf], [d0 + skews[dd], row_ids], vals)
                return carry

            lax.fori_loop(0, _NB // 16, block, 0)
            h = c // n_half
            bpos = b0 + (c - h * n_half) * _NB
            pltpu.async_copy(trans_v.at[buf, :, pl.ds(0, _NB)],
                             out_hbm.at[h, :, pl.ds(bpos, _NB)], osems[buf])

        def drain_out(buf):
            pltpu.make_async_copy(trans_v.at[buf, :, pl.ds(0, _NB)],
                                  out_hbm.at[0, :, pl.ds(b0, _NB)],
                                  osems[buf]).wait()

        fetch(0, 0)

        def body(g, carry):
            c = 2 * g

            @pl.when(g > 0)
            def _():
                drain_out(0)
                drain_out(1)

            @pl.when(c + 1 < n_chunk)
            def _():
                fetch(c + 1, 1)

            drain(0)
            emit(c, 0)

            @pl.when(c + 2 < n_chunk)
            def _():
                fetch(c + 2, 0)

            @pl.when(c + 1 < n_chunk)
            def _():
                drain(1)
                emit(c + 1, 1)

            return carry

        lax.fori_loop(0, (n_chunk + 1) // 2, body, 0)
        drain_out(0)
        drain_out(1)

    return gather_kernel


_RP = 128


@functools.cache
def _make_repack(V: int, D: int):
    nfull = V // 512
    tail = V - nfull * 512
    grid = nfull + (1 if tail else 0)
    n_even = (nfull // _NW) * _NW
    k_max = n_even // _NW
    mesh = plsc.VectorSubcoreMesh(core_axis_name="c", subcore_axis_name="s")

    @functools.partial(
        pl.kernel,
        out_type=jax.ShapeDtypeStruct((grid * 128, 4 * D), jnp.float32),
        mesh=mesh,
        scratch_types=[
            pltpu.VMEM((2, D, 512), jnp.float32),
            pltpu.VMEM((2, 128, _RP), jnp.float32),
            pltpu.VMEM((D, 64), jnp.float32),
            pltpu.SemaphoreType.DMA,
            pltpu.SemaphoreType.DMA,
            pltpu.SemaphoreType.DMA,
            pltpu.SemaphoreType.DMA,
        ],
        compiler_params=pltpu.CompilerParams(needs_layout_passes=False),
    )
    def repack_kernel(tt_hbm, t2_hbm, src2, dst2, src_tail, si0, si1, so0, so1):
        wid = lax.axis_index("s") * 2 + lax.axis_index("c")
        iota16 = lax.iota(jnp.int32, 16)
        isems = (si0, si1)
        osems = (so0, so1)

        def fetch(blk, buf):
            pltpu.async_copy(tt_hbm.at[:, pl.ds(blk * 512, 512)],
                             src2.at[buf], isems[buf])

        def drain_in(buf):
            pltpu.make_async_copy(tt_hbm.at[:, pl.ds(0, 512)],
                                  src2.at[buf], isems[buf]).wait()

        skews = [jnp.bitwise_and(s + iota16, 15) for s in range(16)]

        def transpose(buf):
            def jb_loop(jb, carry):
                j0 = jb * 16
                rows = j0 + iota16
                for q in range(4):
                    for d0 in range(0, D, 16):
                        for s in range(16):
                            dvec = d0 + skews[s]
                            vals = plsc.load_gather(
                                src2.at[buf], [dvec, q * 128 + rows])
                            plsc.store_scatter(
                                dst2.at[buf], [rows, q * D + dvec], vals)
                return carry

            lax.fori_loop(0, 8, jb_loop, 0)

        def fire_out(blk, buf):
            pltpu.async_copy(dst2.at[buf, :, pl.ds(0, 4 * D)],
                             t2_hbm.at[pl.ds(blk * 128, 128)], osems[buf])

        def drain_out(buf):
            pltpu.make_async_copy(dst2.at[buf, :, pl.ds(0, 4 * D)],
                                  t2_hbm.at[pl.ds(0, 128)], osems[buf]).wait()

        fetch(wid, 0)

        def body(g, carry):
            k0 = 2 * g

            @pl.when(g > 0)
            def _():
                drain_out(0)
                drain_out(1)

            fetch(wid + (k0 + 1) * _NW, 1)
            drain_in(0)
            transpose(0)
            fire_out(wid + k0 * _NW, 0)
            fetch(wid + (k0 + 2) * _NW, 0)
            drain_in(1)
            transpose(1)
            fire_out(wid + (k0 + 1) * _NW, 1)
            return carry

        lax.fori_loop(0, (k_max - 1) // 2, body, 0)
        drain_out(0)
        drain_out(1)

        drain_in(0)
        transpose(0)
        fire_out(wid + (k_max - 1) * _NW, 0)
        drain_out(0)

        @pl.when(wid == 0)
        def _():
            pltpu.sync_copy(tt_hbm.at[:, pl.ds(n_even * 512, 512)],
                            src2.at[1])
            transpose(1)
            pltpu.sync_copy(dst2.at[1, :, pl.ds(0, 4 * D)],
                            t2_hbm.at[pl.ds(n_even * 128, 128)])

        if tail:
            @pl.when(wid == 1)
            def _():
                pltpu.sync_copy(tt_hbm.at[:, pl.ds(nfull * 512, tail)],
                                src_tail)
                for jb in range(tail // 16):
                    j0 = jb * 16
                    rows = j0 + iota16
                    for d in range(D):
                        vals = src_tail[d, pl.ds(j0, 16)]
                        plsc.store_scatter(
                            dst2.at[1],
                            [rows, jnp.full((16,), d, jnp.int32)],
                            vals,
                        )
                pltpu.sync_copy(dst2.at[1, pl.ds(0, tail), pl.ds(0, 4 * D)],
                                t2_hbm.at[pl.ds(nfull * 128, tail)])

    return repack_kernel


def kernel(genre_labels, table):
    b, h = genre_labels.shape
    idx2 = genre_labels.T.astype(jnp.int32)
    v, dm = table.shape
    table2 = _make_repack(v, dm)(table.T)
    out = _make_kernel(b, h)(idx2, table2)
    return out.transpose(2, 0, 1)


# speedup vs baseline: 1.0892x; 1.0892x over previous
import functools

import jax
import jax.numpy as jnp
from jax import lax
from jax.experimental import pallas as pl
from jax.experimental.pallas import tpu as pltpu
from jax.experimental.pallas import tpu_sc as plsc

_D = 32
_NW = 32
_NB = 256
_NGI = _NB // 128
_TP = _NB + 2


@functools.cache
def _make_kernel(BATCH: int, HIST: int):
    nb_strip = BATCH // _NW
    n_half = nb_strip // _NB
    n_chunk = HIST * n_half
    mesh = plsc.VectorSubcoreMesh(core_axis_name="c", subcore_axis_name="s")

    @functools.partial(
        pl.kernel,
        out_type=jax.ShapeDtypeStruct((HIST, _D, BATCH), jnp.float32),
        mesh=mesh,
        scratch_types=[
            pltpu.VMEM((HIST, nb_strip), jnp.int32),
            pltpu.VMEM((2 * _NGI, 128), jnp.int32),
            pltpu.VMEM((2, _NB), jnp.int32),
            pltpu.VMEM((2, _NB, 128), jnp.float32),
            pltpu.VMEM((2, _D, _TP), jnp.float32),
            pltpu.SemaphoreType.DMA,
            pltpu.SemaphoreType.DMA,
            pltpu.SemaphoreType.DMA,
            pltpu.SemaphoreType.DMA,
        ],
        compiler_params=pltpu.CompilerParams(needs_layout_passes=False),
    )
    def gather_kernel(idx_hbm, table2_hbm, out_hbm,
                      idx_local, gi, qoff, rows_v, trans_v,
                      sem0, sem1, osem0, osem1):
        wid = lax.axis_index("s") * 2 + lax.axis_index("c")
        b0 = wid * nb_strip
        iota16 = lax.iota(jnp.int32, 16)
        sems = (sem0, sem1)
        osems = (osem0, osem1)

        pltpu.sync_copy(idx_hbm.at[:, pl.ds(b0, nb_strip)], idx_local)

        def fetch(c, buf):
            h = c // n_half
            off = (c - h * n_half) * _NB
            for t in range(_NB // 16):
                v = idx_local[h, pl.ds(off + t * 16, 16)]
                gi[buf * _NGI + t // 8, pl.ds((t % 8) * 16, 16)] = (
                    lax.shift_left(lax.shift_right_logical(v, 9), 7)
                    + jnp.bitwise_and(v, 127))
                qoff[buf, pl.ds(t * 16, 16)] = lax.shift_left(
                    jnp.bitwise_and(lax.shift_right_logical(v, 7), 3), 5)
            for j in range(_NGI):
                pltpu.async_copy(
                    table2_hbm.at[gi.at[buf * _NGI + j]],
                    rows_v.at[buf, pl.ds(j * 128, 128)],
                    sems[buf],
                )

        def drain(buf):
            for j in range(_NGI):
                pltpu.make_async_copy(
                    table2_hbm.at[gi.at[buf * _NGI + j]],
                    rows_v.at[buf, pl.ds(j * 128, 128)],
                    sems[buf],
                ).wait()

        skews = [jnp.bitwise_and(dd + iota16, 15) for dd in range(16)]

        def emit(c, buf):
            def block(bg, carry):
                base = bg * 16
                row_ids = base + iota16
                qv = qoff[buf, pl.ds(base, 16)]
                for d0 in range(0, _D, 16):
                    for dd in range(16):
                        vals = plsc.load_gather(
                            rows_v.at[buf], [row_ids, qv + (d0 + skews[dd])])
                        plsc.store_scatter(
                            trans_v.at[buf], [d0 + skews[dd], row_ids], vals)
                return carry

            lax.fori_loop(0, _NB // 16, block, 0)
            h = c // n_half
            bpos = b0 + (c - h * n_half) * _NB
            pltpu.async_copy(trans_v.at[buf, :, pl.ds(0, _NB)],
                             out_hbm.at[h, :, pl.ds(bpos, _NB)], osems[buf])

        def drain_out(buf):
            pltpu.make_async_copy(trans_v.at[buf, :, pl.ds(0, _NB)],
                                  out_hbm.at[0, :, pl.ds(b0, _NB)],
                                  osems[buf]).wait()

        fetch(0, 0)

        def body(g, carry):
            c = 2 * g

            @pl.when(g > 0)
            def _():
                drain_out(0)
                drain_out(1)

            @pl.when(c + 1 < n_chunk)
            def _():
                fetch(c + 1, 1)

            drain(0)
            emit(c, 0)

            @pl.when(c + 2 < n_chunk)
            def _():
                fetch(c + 2, 0)

            @pl.when(c + 1 < n_chunk)
            def _():
                drain(1)
                emit(c + 1, 1)

            return carry

        lax.fori_loop(0, (n_chunk + 1) // 2, body, 0)
        drain_out(0)
        drain_out(1)

    return gather_kernel


_RP = 128


@functools.cache
def _make_repack(V: int, D: int):
    nfull = V // 512
    tail = V - nfull * 512
    grid = nfull + (1 if tail else 0)
    n_even = (nfull // _NW) * _NW
    k_max = n_even // _NW
    mesh = plsc.VectorSubcoreMesh(core_axis_name="c", subcore_axis_name="s")

    @functools.partial(
        pl.kernel,
        out_type=jax.ShapeDtypeStruct((grid * 128, 4 * D), jnp.float32),
        mesh=mesh,
        scratch_types=[
            pltpu.VMEM((2, D, 512), jnp.float32),
            pltpu.VMEM((2, 128, _RP), jnp.float32),
            pltpu.VMEM((D, 64), jnp.float32),
            pltpu.SemaphoreType.DMA,
            pltpu.SemaphoreType.DMA,
            pltpu.SemaphoreType.DMA,
            pltpu.SemaphoreType.DMA,
        ],
        compiler_params=pltpu.CompilerParams(needs_layout_passes=False),
    )
    def repack_kernel(tt_hbm, t2_hbm, src2, dst2, src_tail, si0, si1, so0, so1):
        wid = lax.axis_index("s") * 2 + lax.axis_index("c")
        iota16 = lax.iota(jnp.int32, 16)
        isems = (si0, si1)
        osems = (so0, so1)

        def fetch(blk, buf):
            pltpu.async_copy(tt_hbm.at[:, pl.ds(blk * 512, 512)],
                             src2.at[buf], isems[buf])

        def drain_in(buf):
            pltpu.make_async_copy(tt_hbm.at[:, pl.ds(0, 512)],
                                  src2.at[buf], isems[buf]).wait()

        skews = [jnp.bitwise_and(s + iota16, 15) for s in range(16)]

        def transpose(buf):
            def jb_loop(jb, carry):
                j0 = jb * 16
                rows = j0 + iota16
                for q in range(4):
                    for d0 in range(0, D, 16):
                        for s in range(16):
                            dvec = d0 + skews[s]
                            vals = plsc.load_gather(
                                src2.at[buf], [dvec, q * 128 + rows])
                            plsc.store_scatter(
                                dst2.at[buf], [rows, q * D + dvec], vals)
                return carry

            lax.fori_loop(0, 8, jb_loop, 0)

        def fire_out(blk, buf):
            pltpu.async_copy(dst2.at[buf, :, pl.ds(0, 4 * D)],
                             t2_hbm.at[pl.ds(blk * 128, 128)], osems[buf])

        def drain_out(buf):
            pltpu.make_async_copy(dst2.at[buf, :, pl.ds(0, 4 * D)],
                                  t2_hbm.at[pl.ds(0, 128)], osems[buf]).wait()

        fetch(wid, 0)

        def body(g, carry):
            k0 = 2 * g
            fetch(wid + (k0 + 1) * _NW, 1)
            drain_in(0)
            transpose(0)
            fire_out(wid + k0 * _NW, 0)
            fetch(wid + (k0 + 2) * _NW, 0)
            drain_in(1)
            transpose(1)
            fire_out(wid + (k0 + 1) * _NW, 1)
            drain_out(0)
            drain_out(1)
            return carry

        lax.fori_loop(0, (k_max - 1) // 2, body, 0)

        drain_in(0)
        transpose(0)
        fire_out(wid + (k_max - 1) * _NW, 0)
        drain_out(0)

        @pl.when(wid == 0)
        def _():
            pltpu.sync_copy(tt_hbm.at[:, pl.ds(n_even * 512, 512)],
                            src2.at[1])
            transpose(1)
            pltpu.sync_copy(dst2.at[1, :, pl.ds(0, 4 * D)],
                            t2_hbm.at[pl.ds(n_even * 128, 128)])

        if tail:
            @pl.when(wid == 1)
            def _():
                pltpu.sync_copy(tt_hbm.at[:, pl.ds(nfull * 512, tail)],
                                src_tail)
                for jb in range(tail // 16):
                    j0 = jb * 16
                    rows = j0 + iota16
                    for d in range(D):
                        vals = src_tail[d, pl.ds(j0, 16)]
                        plsc.store_scatter(
                            dst2.at[1],
                            [rows, jnp.full((16,), d, jnp.int32)],
                            vals,
                        )
                pltpu.sync_copy(dst2.at[1, pl.ds(0, tail), pl.ds(0, 4 * D)],
                                t2_hbm.at[pl.ds(nfull * 128, tail)])

    return repack_kernel


def kernel(genre_labels, table):
    b, h = genre_labels.shape
    idx2 = genre_labels.T.astype(jnp.int32)
    v, dm = table.shape
    table2 = _make_repack(v, dm)(table.T)
    out = _make_kernel(b, h)(idx2, table2)
    return out.transpose(2, 0, 1)
